# baseline (speedup 1.0000x reference)
import functools

import jax
import jax.numpy as jnp
import numpy as np
from jax import lax
from jax.experimental import pallas as pl
from jax.experimental.pallas import tpu as pltpu
from jax.experimental.pallas import tpu_sc as plsc

NC = 2
NS = 16
NW = NC * NS
LANES = 16
ROWS_PER_STREAM = 128
SPB = 2
NBUF = 2


def _build_sc_call(n_pad, deg, emb, npw):
    nodes_per_stream = ROWS_PER_STREAM // deg
    nodes_per_buf = nodes_per_stream * SPB
    nstream = (npw * deg) // ROWS_PER_STREAM
    nbl = nstream // SPB
    ngroup = nbl // NBUF
    nvec = emb // LANES
    inv_deg = np.float32(1.0 / deg)
    i32 = np.int32

    mesh = plsc.VectorSubcoreMesh(
        core_axis_name="c", subcore_axis_name="s",
        num_cores=NC, num_subcores=NS)

    @functools.partial(
        pl.kernel,
        out_type=jax.ShapeDtypeStruct((n_pad, emb), jnp.float32),
        mesh=mesh,
        scratch_types=(
            [pltpu.VMEM((nstream, ROWS_PER_STREAM), jnp.int32),
             pltpu.VMEM((npw, emb), jnp.float32)]
            + [pltpu.VMEM((SPB * ROWS_PER_STREAM, emb), jnp.float32)]
            * NBUF
            + [pltpu.SemaphoreType.DMA] * NBUF
        ),
    )
    def sc_call(idx_hbm, table_hbm, out_hbm, idx_v, out_v, *rest):
        bufs, sems = rest[:NBUF], rest[NBUF:]
        w = lax.axis_index("s") * i32(NC) + lax.axis_index("c")

        pltpu.sync_copy(idx_hbm.at[w], idx_v)

        def buf_start(bl, b):
            for k in range(SPB):
                pltpu.async_copy(
                    table_hbm.at[idx_v.at[bl * i32(SPB) + i32(k)]],
                    bufs[b].at[pl.ds(k * ROWS_PER_STREAM,
                                     ROWS_PER_STREAM)],
                    sems[b])

        def buf_wait(bl, b):
            for k in range(SPB):
                pltpu.make_async_copy(
                    table_hbm.at[idx_v.at[bl * i32(SPB) + i32(k)]],
                    bufs[b].at[pl.ds(k * ROWS_PER_STREAM,
                                     ROWS_PER_STREAM)],
                    sems[b]).wait()

        for b in range(NBUF):
            buf_start(jnp.int32(b), b)

        def group_body(g, carry):
            for b in range(NBUF):
                bl = g * i32(NBUF) + i32(b)
                buf_wait(bl, b)
                buf = bufs[b]

                def node_body(n, c, buf=buf, bl=bl):
                    r = bl * i32(nodes_per_buf) + n
                    base = n * i32(deg)
                    for v0 in range(0, nvec, 4):
                        sls = [pl.ds(v * LANES, LANES)
                               for v in range(v0, v0 + 4)]
                        accs = [buf[base, sl] for sl in sls]
                        for d in range(1, deg):
                            row = base + i32(d)
                            for k in range(4):
                                accs[k] = accs[k] + buf[row, sls[k]]
                        for k in range(4):
                            out_v[r, sls[k]] = accs[k] * inv_deg
                    return c

                lax.fori_loop(i32(0), i32(nodes_per_buf), node_body, 0)

                @pl.when(bl + i32(NBUF) < i32(nbl))
                def _(bl=bl, b=b):
                    buf_start(bl + i32(NBUF), b)
            return carry

        lax.fori_loop(i32(0), i32(ngroup), group_body, 0)

        pltpu.sync_copy(out_v, out_hbm.at[pl.ds(w * i32(npw), npw)])

    return sc_call


def kernel(nodes, to_neighs, u2e_weight):
    del nodes
    n, deg = to_neighs.shape
    emb = u2e_weight.shape[1]

    nodes_per_stream = ROWS_PER_STREAM // deg
    quantum = nodes_per_stream * SPB * NBUF
    npw = ((n + NW - 1) // NW + quantum - 1) // quantum * quantum
    n_pad = npw * NW

    with jax.enable_x64(False):
        idx = to_neighs.astype(jnp.int32).reshape(-1)
        idx = jnp.pad(idx, (0, n_pad * deg - n * deg))
        idx3 = idx.reshape(NW, (npw * deg) // ROWS_PER_STREAM,
                           ROWS_PER_STREAM)

        table = u2e_weight.astype(jnp.float32)
        sc_call = _build_sc_call(n_pad, deg, emb, npw)
        out = sc_call(idx3, table)
        return out[:n]


# speedup vs baseline: 1.0186x; 1.0186x over previous
import functools

import jax
import jax.numpy as jnp
import numpy as np
from jax import lax
from jax.experimental import pallas as pl
from jax.experimental.pallas import tpu as pltpu
from jax.experimental.pallas import tpu_sc as plsc

NC = 2
NS = 16
NW = NC * NS
LANES = 16
ROWS_PER_STREAM = 128
NBUF = 4


def _build_sc_call(n_pad, deg, emb, npw):
    nodes_per_stream = ROWS_PER_STREAM // deg
    nchunk = (npw * deg) // ROWS_PER_STREAM
    ngroup = nchunk // NBUF
    nvec = emb // LANES
    inv_deg = np.float32(1.0 / deg)
    i32 = np.int32

    mesh = plsc.VectorSubcoreMesh(
        core_axis_name="c", subcore_axis_name="s",
        num_cores=NC, num_subcores=NS)

    @functools.partial(
        pl.kernel,
        out_type=jax.ShapeDtypeStruct((n_pad, emb), jnp.float32),
        mesh=mesh,
        scratch_types=(
            [pltpu.VMEM((nchunk, ROWS_PER_STREAM), jnp.int32),
             pltpu.VMEM((npw, emb), jnp.float32)]
            + [pltpu.VMEM((ROWS_PER_STREAM, emb), jnp.float32)] * NBUF
            + [pltpu.SemaphoreType.DMA] * NBUF
        ),
    )
    def sc_call(idx_hbm, table_hbm, out_hbm, idx_v, out_v, *rest):
        bufs, sems = rest[:NBUF], rest[NBUF:]
        w = lax.axis_index("s") * i32(NC) + lax.axis_index("c")

        pltpu.sync_copy(idx_hbm.at[w], idx_v)

        def gather_start(j, b):
            pltpu.async_copy(table_hbm.at[idx_v.at[j]], bufs[b], sems[b])

        def gather_wait(j, b):
            pltpu.make_async_copy(
                table_hbm.at[idx_v.at[j]], bufs[b], sems[b]).wait()

        for b in range(NBUF):
            gather_start(jnp.int32(b), b)

        def group_body(g, carry):
            for b in range(NBUF):
                j = g * i32(NBUF) + i32(b)
                gather_wait(j, b)
                buf = bufs[b]

                def node_body(n, c, buf=buf, j=j):
                    r = j * i32(nodes_per_stream) + n
                    base = n * i32(deg)
                    for v0 in range(0, nvec, 4):
                        sls = [pl.ds(v * LANES, LANES)
                               for v in range(v0, v0 + 4)]
                        accs = [buf[base, sl] for sl in sls]
                        for d in range(1, deg):
                            row = base + i32(d)
                            for k in range(4):
                                accs[k] = accs[k] + buf[row, sls[k]]
                        for k in range(4):
                            out_v[r, sls[k]] = accs[k] * inv_deg
                    return c

                lax.fori_loop(i32(0), i32(nodes_per_stream), node_body, 0)

                @pl.when(j + i32(NBUF) < i32(nchunk))
                def _(j=j, b=b):
                    gather_start(j + i32(NBUF), b)
            return carry

        lax.fori_loop(i32(0), i32(ngroup), group_body, 0)

        pltpu.sync_copy(out_v, out_hbm.at[pl.ds(w * i32(npw), npw)])

    return sc_call


def kernel(nodes, to_neighs, u2e_weight):
    del nodes
    n, deg = to_neighs.shape
    emb = u2e_weight.shape[1]

    nodes_per_stream = ROWS_PER_STREAM // deg
    quantum = nodes_per_stream * NBUF
    npw = ((n + NW - 1) // NW + quantum - 1) // quantum * quantum
    n_pad = npw * NW

    with jax.enable_x64(False):
        idx = to_neighs.astype(jnp.int32).reshape(-1)
        idx = jnp.pad(idx, (0, n_pad * deg - n * deg))
        idx3 = idx.reshape(NW, (npw * deg) // ROWS_PER_STREAM,
                           ROWS_PER_STREAM)

        table = u2e_weight.astype(jnp.float32)
        sc_call = _build_sc_call(n_pad, deg, emb, npw)
        out = sc_call(idx3, table)
        return out[:n]
